# baseline (speedup 1.0000x reference)
import functools

import jax
import jax.numpy as jnp
from jax import lax
from jax.experimental import pallas as pl
from jax.experimental.pallas import tpu as pltpu
from jax.experimental.pallas import tpu_sc as plsc

_N = 16384
_C = 1000
_NC = 2
_NS = 16
_NW = _NC * _NS
_RPW = _N // _NW
_LANES = 16
_CHUNK = 128


def _sc_body(value_hbm, idx_hbm, out_hbm, idx_v, flat_v, gath_v, sem):
    wid = lax.axis_index("s") * _NC + lax.axis_index("c")
    base = wid * _RPW

    pltpu.sync_copy(idx_hbm.at[pl.ds(base, _RPW)], idx_v)

    lane = lax.iota(jnp.int32, _LANES)
    for i in range(_RPW // _LANES):
        rows = (base + i * _LANES) + lane
        flat_v[pl.ds(i * _LANES, _LANES)] = (
            idx_v[pl.ds(i * _LANES, _LANES)] + rows * _C
        )

    copies = [
        pltpu.async_copy(
            value_hbm.at[flat_v.at[pl.ds(c * _CHUNK, _CHUNK)]],
            gath_v.at[pl.ds(c * _CHUNK, _CHUNK)],
            sem,
        )
        for c in range(_RPW // _CHUNK)
    ]
    for cp in copies:
        cp.wait()

    pltpu.sync_copy(gath_v, out_hbm.at[pl.ds(base, _RPW)])


@jax.jit
def kernel(value, index):
    mesh = plsc.VectorSubcoreMesh(core_axis_name="c", subcore_axis_name="s")
    run = functools.partial(
        pl.kernel,
        out_type=jax.ShapeDtypeStruct((_N,), jnp.float32),
        mesh=mesh,
        scratch_types=[
            pltpu.VMEM((_RPW,), jnp.int32),
            pltpu.VMEM((_RPW,), jnp.int32),
            pltpu.VMEM((_RPW,), jnp.float32),
            pltpu.SemaphoreType.DMA,
        ],
    )(_sc_body)
    flat = run(value.reshape(_N * _C), index.reshape(_N).astype(jnp.int32))
    return flat.reshape(_N, 1)


# speedup vs baseline: 1.6788x; 1.6788x over previous
import functools

import jax
import jax.numpy as jnp
from jax import lax
from jax.experimental import pallas as pl
from jax.experimental.pallas import tpu as pltpu
from jax.experimental.pallas import tpu_sc as plsc

_N = 16384
_C = 1000
_NC = 2
_NS = 16
_NW = _NC * _NS
_RPW = _N // _NW
_HALF = _RPW // 2
_LANES = 16
_TILE_W = 128
_NFB = _C // _TILE_W
_TAIL = _NFB * _TILE_W
_TAIL_W = _C - _TAIL
_CHUNK = _LANES
_LSLOTS = _HALF + _NFB * (_CHUNK - 1)


def _sc_body(value_hbm, idx_hbm, out_hbm,
             idx_v, bkt_v, lines_v, tail_v, out_v, sem, tail_sem):
    wid = lax.axis_index("s") * _NC + lax.axis_index("c")
    base = wid * _RPW

    pltpu.sync_copy(idx_hbm.at[pl.ds(base, _RPW)], idx_v)

    lane = lax.iota(jnp.int32, _LANES)
    nchunks = _HALF // _LANES

    for p in range(2):
        pbase = base + p * _HALF
        poff = p * _HALF

        tail_cp = pltpu.async_copy(
            value_hbm.at[pl.ds(pbase, _HALF), pl.ds(_TAIL, _TAIL_W)],
            tail_v, tail_sem)

        fill = jnp.full((_LANES,), pbase, jnp.int32)

        def _prefill(i, _):
            bkt_v[pl.ds(i * _LANES, _LANES)] = fill
            return 0

        lax.fori_loop(0, _NFB * _HALF // _LANES, _prefill, 0)

        def _build(k, cnts):
            off = pl.multiple_of(k * _LANES, _LANES)
            idxc = idx_v[pl.ds(poff + off, _LANES)]
            cb = lax.shift_right_logical(idxc, 7)
            rows = (pbase + off) + lane
            new = []
            for b in range(_NFB):
                mask = cb == b
                mi = mask.astype(jnp.int32)
                pos = cnts[b] + plsc.cumsum(mi) - 1
                plsc.store_scatter(bkt_v, [pos + b * _HALF], rows, mask=mask)
                new.append(cnts[b] + jnp.sum(mi))
            return tuple(new)

        cnts = lax.fori_loop(0, nchunks, _build, (jnp.int32(0),) * _NFB)

        starts = []
        nstreams = []
        pstart = jnp.int32(0)
        for b in range(_NFB):
            ns = lax.div(cnts[b] + (_CHUNK - 1), _CHUNK)
            starts.append(pstart)
            nstreams.append(ns)
            dst0 = pstart

            def _fire(c, _, b=b, dst0=dst0):
                lsrc = pl.multiple_of(b * _HALF + c * _CHUNK, _CHUNK)
                ldst = pl.multiple_of(dst0 + c * _CHUNK, _CHUNK)
                pltpu.make_async_copy(
                    value_hbm.at[
                        bkt_v.at[pl.ds(lsrc, _CHUNK)],
                        pl.ds(b * _TILE_W, _TILE_W),
                    ],
                    lines_v.at[pl.ds(ldst, _CHUNK), :],
                    sem,
                ).start()
                return 0

            lax.fori_loop(0, ns, _fire, 0)
            pstart = pstart + ns * _CHUNK

        tail_cp.wait()

        def _tail_extract(k, _):
            off = pl.multiple_of(k * _LANES, _LANES)
            rowk = lane + off
            idxc = idx_v[pl.ds(poff + off, _LANES)]
            in_tail = idxc >= _TAIL
            col_t = jnp.clip(idxc - _TAIL, 0, _TAIL_W - 1)
            tl = plsc.load_gather(tail_v, [rowk, col_t])
            out_v[pl.ds(poff + off, _LANES)] = jnp.where(in_tail, tl, 0.0)
            return 0

        lax.fori_loop(0, nchunks, _tail_extract, 0)

        total = nstreams[0]
        for b in range(1, _NFB):
            total = total + nstreams[b]

        def _drain(c, _):
            pltpu.make_async_copy(
                value_hbm.at[
                    bkt_v.at[pl.ds(0, _CHUNK)],
                    pl.ds(0, _TILE_W),
                ],
                lines_v.at[pl.ds(0, _CHUNK), :],
                sem,
            ).wait()
            return 0

        lax.fori_loop(0, total, _drain, 0)

        for b in range(_NFB):
            def _extract(q, _, b=b, dst0=starts[b]):
                gpos = dst0 + q * _CHUNK + lane
                lpos = pl.multiple_of(b * _HALF + q * _CHUNK, _CHUNK)
                rows = bkt_v[pl.ds(lpos, _CHUNK)]
                local = rows - base
                cols = plsc.load_gather(idx_v, [local]) - b * _TILE_W
                cols = jnp.clip(cols, 0, _TILE_W - 1)
                vals = plsc.load_gather(lines_v, [gpos, cols])
                mask = (q * _CHUNK + lane) < cnts[b]
                plsc.store_scatter(out_v, [local], vals, mask=mask)
                return 0

            lax.fori_loop(0, nstreams[b], _extract, 0)

    pltpu.sync_copy(out_v, out_hbm.at[pl.ds(base, _RPW)])


@jax.jit
def kernel(value, index):
    mesh = plsc.VectorSubcoreMesh(core_axis_name="c", subcore_axis_name="s")
    run = functools.partial(
        pl.kernel,
        out_type=jax.ShapeDtypeStruct((_N,), jnp.float32),
        mesh=mesh,
        compiler_params=pltpu.CompilerParams(needs_layout_passes=False),
        scratch_types=[
            pltpu.VMEM((_RPW,), jnp.int32),
            pltpu.VMEM((_NFB * _HALF,), jnp.int32),
            pltpu.VMEM((_LSLOTS, _TILE_W), jnp.float32),
            pltpu.VMEM((_HALF, _TAIL_W), jnp.float32),
            pltpu.VMEM((_RPW,), jnp.float32),
            pltpu.SemaphoreType.DMA,
            pltpu.SemaphoreType.DMA,
        ],
    )(_sc_body)
    flat = run(value, index.reshape(_N).astype(jnp.int32))
    return flat.reshape(_N, 1)


# speedup vs baseline: 5.2240x; 3.1117x over previous
import functools

import jax
import jax.numpy as jnp
from jax import lax
from jax.experimental import pallas as pl
from jax.experimental.pallas import tpu as pltpu
from jax.experimental.pallas import tpu_sc as plsc

_N = 16384
_C = 1000
_NC = 2
_NS = 16
_NW = _NC * _NS
_RPW = _N // _NW
_LANES = 16
_TILE_W = 128
_CHUNK = 128


def _sc_body(vt_hbm, idx_hbm, out_hbm, idx_v, lines_v, out_v, sem):
    wid = lax.axis_index("s") * _NC + lax.axis_index("c")
    base = wid * _RPW

    pltpu.sync_copy(idx_hbm.at[pl.ds(base, _RPW)], idx_v)

    copies = []
    for c in range(_RPW // _CHUNK):
        win = pl.multiple_of(base + c * _CHUNK, _TILE_W)
        copies.append(pltpu.async_copy(
            vt_hbm.at[idx_v.at[pl.ds(c * _CHUNK, _CHUNK)],
                      pl.ds(win, _TILE_W)],
            lines_v.at[pl.ds(c * _CHUNK, _CHUNK), :],
            sem,
        ))
    for cp in copies:
        cp.wait()

    lane = lax.iota(jnp.int32, _LANES)
    for k in range(_RPW // _LANES):
        o = lane + k * _LANES
        col = jnp.bitwise_and(o, _TILE_W - 1)
        out_v[pl.ds(k * _LANES, _LANES)] = plsc.load_gather(lines_v, [o, col])

    pltpu.sync_copy(out_v, out_hbm.at[pl.ds(base, _RPW)])


@jax.jit
def kernel(value, index):
    mesh = plsc.VectorSubcoreMesh(core_axis_name="c", subcore_axis_name="s")
    run = functools.partial(
        pl.kernel,
        out_type=jax.ShapeDtypeStruct((_N,), jnp.float32),
        mesh=mesh,
        compiler_params=pltpu.CompilerParams(needs_layout_passes=False),
        scratch_types=[
            pltpu.VMEM((_RPW,), jnp.int32),
            pltpu.VMEM((_RPW, _TILE_W), jnp.float32),
            pltpu.VMEM((_RPW,), jnp.float32),
            pltpu.SemaphoreType.DMA,
        ],
    )(_sc_body)
    flat = run(value.T, index.reshape(_N).astype(jnp.int32))
    return flat.reshape(_N, 1)


# speedup vs baseline: 5.3021x; 1.0149x over previous
import functools

import jax
import jax.numpy as jnp
from jax import lax
from jax.experimental import pallas as pl
from jax.experimental.pallas import tpu as pltpu
from jax.experimental.pallas import tpu_sc as plsc

_N = 16384
_C = 1000
_NC = 2
_NS = 16
_NW = _NC * _NS
_RPW = _N // _NW
_LANES = 16
_TILE_W = 128
_CHUNK = 128


def _sc_body(vt_hbm, idx_hbm, out_hbm, idx_v, lines_v, out_v, sem):
    wid = lax.axis_index("s") * _NC + lax.axis_index("c")
    base = wid * _RPW

    pltpu.sync_copy(idx_hbm.at[pl.ds(base, _RPW)], idx_v)

    copies = []
    for c in range(_RPW // _CHUNK):
        win = pl.multiple_of(base + c * _CHUNK, _TILE_W)
        copies.append(pltpu.async_copy(
            vt_hbm.at[idx_v.at[pl.ds(c * _CHUNK, _CHUNK)],
                      pl.ds(win, _TILE_W)],
            lines_v.at[pl.ds(c * _CHUNK, _CHUNK), :],
            sem,
        ))
    for cp in copies:
        cp.wait()

    lane = lax.iota(jnp.int32, _LANES)
    for k in range(_RPW // _LANES):
        o = lane + k * _LANES
        col = jnp.bitwise_and(o, _TILE_W - 1)
        out_v[pl.ds(k * _LANES, _LANES)] = plsc.load_gather(lines_v, [o, col])

    pltpu.sync_copy(out_v, out_hbm.at[pl.ds(base, _RPW)])


@jax.jit
def kernel(value, index):
    mesh = plsc.VectorSubcoreMesh(core_axis_name="c", subcore_axis_name="s")
    run = functools.partial(
        pl.kernel,
        out_type=jax.ShapeDtypeStruct((_N,), jnp.float32),
        mesh=mesh,
        compiler_params=pltpu.CompilerParams(
            needs_layout_passes=False,
            skip_device_barrier=True,
            disable_semaphore_checks=True,
            disable_bounds_checks=True,
        ),
        scratch_types=[
            pltpu.VMEM((_RPW,), jnp.int32),
            pltpu.VMEM((_RPW, _TILE_W), jnp.float32),
            pltpu.VMEM((_RPW,), jnp.float32),
            pltpu.SemaphoreType.DMA,
        ],
    )(_sc_body)
    flat = run(value.T, index.reshape(_N).astype(jnp.int32))
    return flat.reshape(_N, 1)
